# baseline (speedup 1.0000x reference)
import functools

import jax
import jax.numpy as jnp
from jax import lax
from jax.experimental import pallas as pl
from jax.experimental.pallas import tpu as pltpu
from jax.experimental.pallas import tpu_sc as plsc

H = 224
W = 224
L = H * W
C = 96
OC = 384
KNN = 5
NSEL = 9
PAD = KNN // 2
ZERO_ROW = L


def _shift_plane(a, dy, dx):
    b = a
    if dy > 0:
        b = jnp.concatenate([b[dy:, :], jnp.zeros((dy, W), b.dtype)], axis=0)
    elif dy < 0:
        b = jnp.concatenate([jnp.zeros((-dy, W), b.dtype), b[:H + dy, :]],
                            axis=0)
    if dx > 0:
        b = jnp.concatenate([b[:, dx:], jnp.zeros((H, dx), b.dtype)], axis=1)
    elif dx < 0:
        b = jnp.concatenate([jnp.zeros((H, -dx), b.dtype), b[:, :W + dx]],
                            axis=1)
    return b


def _topk_kernel(ch0_ref, ms_ref, idx_ref):
    mean = ms_ref[0, 0]
    std = ms_ref[0, 1]
    norm = (ch0_ref[...] * std + mean).astype(jnp.bfloat16).astype(
        jnp.float32)

    ys = lax.broadcasted_iota(jnp.int32, (H, W), 0)
    xs = lax.broadcasted_iota(jnp.int32, (H, W), 1)

    diffs = []
    lins = []
    for ky in range(KNN):
        for kx in range(KNN):
            dy, dx = ky - PAD, kx - PAD
            v = _shift_plane(norm, dy, dx)
            d = jnp.abs(v - norm)
            if dy == 0 and dx == 0:
                d = jnp.full((H, W), -1.0, dtype=jnp.float32)
            diffs.append(d)
            yy = ys + dy
            xx = xs + dx
            ok = (yy >= 0) & (yy < H) & (xx >= 0) & (xx < W)
            lins.append(jnp.where(ok, yy * W + xx, ZERO_ROW))

    big = jnp.float32(jnp.inf)
    for j in range(NSEL):
        m = functools.reduce(jnp.minimum, diffs)
        found = jnp.zeros((H, W), dtype=jnp.bool_)
        idx_j = jnp.full((H, W), ZERO_ROW, dtype=jnp.int32)
        for i in range(KNN * KNN):
            take = jnp.logical_and(jnp.logical_not(found), diffs[i] == m)
            found = jnp.logical_or(found, take)
            idx_j = jnp.where(take, lins[i], idx_j)
            diffs[i] = jnp.where(take, big, diffs[i])
        idx_ref[j, :, :] = idx_j


def _topk_indices(ch0, mean, std):
    ms = jnp.stack([mean[0], std[0]]).reshape(1, 2)
    return pl.pallas_call(
        _topk_kernel,
        out_shape=jax.ShapeDtypeStruct((NSEL, H, W), jnp.int32),
        in_specs=[
            pl.BlockSpec(memory_space=pltpu.VMEM),
            pl.BlockSpec(memory_space=pltpu.SMEM),
        ],
        out_specs=pl.BlockSpec(memory_space=pltpu.VMEM),
    )(ch0, ms)


TT = 512
TROWS = L + TT
NTT = TROWS // TT


def _transpose_kernel(x_ref, t_ref):
    i = pl.program_id(0)

    @pl.when(i < NTT - 1)
    def _():
        t_ref[...] = x_ref[...].T.astype(jnp.bfloat16)

    @pl.when(i == NTT - 1)
    def _():
        t_ref[...] = jnp.zeros((TT, C), jnp.bfloat16)


def _make_table(x2d):
    return pl.pallas_call(
        _transpose_kernel,
        grid=(NTT,),
        in_specs=[
            pl.BlockSpec((C, TT), lambda i: (0, jnp.minimum(i, NTT - 2))),
        ],
        out_specs=pl.BlockSpec((TT, C), lambda i: (i, 0)),
        out_shape=jax.ShapeDtypeStruct((TROWS, C), jnp.bfloat16),
    )(x2d)


NSPLIT = 4
QRT = L // NSPLIT
NROWS_Q = QRT * NSEL
NCORES = 2
NSUB = 16
NWORKERS = NCORES * NSUB
ROWS_PER_W = NROWS_Q // NWORKERS
CHUNK = 504
NCHUNK = ROWS_PER_W // CHUNK


def _gather_body(table_hbm, idx_hbm, out_hbm, idx_v, rows0, rows1,
                 sem0, sem1):
    wid = lax.axis_index("s") * NCORES + lax.axis_index("c")
    base = wid * ROWS_PER_W
    pltpu.sync_copy(idx_hbm.at[pl.ds(base, ROWS_PER_W)], idx_v)
    bufs = (rows0, rows1)
    sems = (sem0, sem1)

    def start(k, b):
        pltpu.async_copy(
            table_hbm.at[idx_v.at[pl.ds(k * CHUNK, CHUNK)]], bufs[b],
            sems[b])

    def wait(b):
        pltpu.make_async_copy(
            table_hbm.at[idx_v.at[pl.ds(0, CHUNK)]], bufs[b],
            sems[b]).wait()

    def put(k, b):
        pltpu.sync_copy(bufs[b], out_hbm.at[pl.ds(base + k * CHUNK, CHUNK)])

    start(0, 0)

    def pair(p, carry):
        k0 = 2 * p

        start(k0 + 1, 1)
        wait(0)
        put(k0, 0)

        @pl.when(k0 + 2 < NCHUNK)
        def _():
            start(k0 + 2, 0)

        wait(1)
        put(k0 + 1, 1)
        return carry

    lax.fori_loop(0, NCHUNK // 2, pair, 0)
    if NCHUNK % 2 == 1:
        wait(0)
        put(NCHUNK - 1, 0)


@functools.cache
def _make_sc_gather():
    return pl.kernel(
        _gather_body,
        out_type=jax.ShapeDtypeStruct((NROWS_Q, C), jnp.bfloat16),
        mesh=plsc.VectorSubcoreMesh(core_axis_name="c", subcore_axis_name="s"),
        scratch_types=[
            pltpu.VMEM((ROWS_PER_W,), jnp.int32),
            pltpu.VMEM((CHUNK, C), jnp.bfloat16),
            pltpu.VMEM((CHUNK, C), jnp.bfloat16),
            pltpu.SemaphoreType.DMA,
            pltpu.SemaphoreType.DMA,
        ],
        compiler_params=pltpu.CompilerParams(use_tc_tiling_on_sc=False),
    )


LT = 448
NT = L // LT
NTQ = NT // NSPLIT


def _matmul_kernel(g1_ref, g2_ref, g3_ref, g4_ref, w_ref, out_ref):
    i = pl.program_id(0)
    grefs = (g1_ref, g2_ref, g3_ref, g4_ref)
    for q in range(NSPLIT):
        @pl.when((i >= q * NTQ) & (i < (q + 1) * NTQ))
        def _(q=q):
            out_ref[...] = jnp.dot(grefs[q][...], w_ref[...],
                                   preferred_element_type=jnp.float32)


def _matmul(gs, w2t_bf16):
    def gspec(q):
        return pl.BlockSpec(
            (LT, NSEL * C),
            lambda i: (jnp.clip(i - q * NTQ, 0, NTQ - 1), 0))

    return pl.pallas_call(
        _matmul_kernel,
        grid=(NT,),
        in_specs=[gspec(0), gspec(1), gspec(2), gspec(3),
                  pl.BlockSpec((NSEL * C, OC), lambda i: (0, 0))],
        out_specs=pl.BlockSpec((LT, OC), lambda i: (i, 0)),
        out_shape=jax.ShapeDtypeStruct((L, OC), jnp.float32),
    )(*gs, w2t_bf16)


def kernel(input, mean, std, pconv_weight):
    x = input.reshape(C, L)
    table = _make_table(x)

    idx9 = _topk_indices(input[0, 0], mean, std)
    idx9 = idx9.reshape(NSEL, L)
    gather = _make_sc_gather()
    gs = []
    for q in range(NSPLIT):
        fi = idx9[:, q * QRT:(q + 1) * QRT].T.reshape(NROWS_Q)
        gs.append(gather(table, fi).reshape(QRT, NSEL * C))

    w_flat = pconv_weight.reshape(OC, C * NSEL)
    w2t = w_flat.reshape(OC, C, NSEL).transpose(2, 1, 0).reshape(NSEL * C, OC)
    out_rows = _matmul(gs, w2t.astype(jnp.bfloat16))
    return out_rows.T.reshape(1, OC, H, W)


# speedup vs baseline: 1.0039x; 1.0039x over previous
import functools

import jax
import jax.numpy as jnp
from jax import lax
from jax.experimental import pallas as pl
from jax.experimental.pallas import tpu as pltpu
from jax.experimental.pallas import tpu_sc as plsc

H = 224
W = 224
L = H * W
C = 96
OC = 384
KNN = 5
NSEL = 9
PAD = KNN // 2
ZERO_ROW = L


def _shift_plane(a, dy, dx):
    b = a
    if dy > 0:
        b = jnp.concatenate([b[dy:, :], jnp.zeros((dy, W), b.dtype)], axis=0)
    elif dy < 0:
        b = jnp.concatenate([jnp.zeros((-dy, W), b.dtype), b[:H + dy, :]],
                            axis=0)
    if dx > 0:
        b = jnp.concatenate([b[:, dx:], jnp.zeros((H, dx), b.dtype)], axis=1)
    elif dx < 0:
        b = jnp.concatenate([jnp.zeros((H, -dx), b.dtype), b[:, :W + dx]],
                            axis=1)
    return b


def _topk_kernel(ch0_ref, ms_ref, idx_ref):
    mean = ms_ref[0, 0]
    std = ms_ref[0, 1]
    norm = (ch0_ref[...] * std + mean).astype(jnp.bfloat16).astype(
        jnp.float32)

    ys = lax.broadcasted_iota(jnp.int32, (H, W), 0)
    xs = lax.broadcasted_iota(jnp.int32, (H, W), 1)

    diffs = []
    lins = []
    for ky in range(KNN):
        for kx in range(KNN):
            dy, dx = ky - PAD, kx - PAD
            v = _shift_plane(norm, dy, dx)
            d = jnp.abs(v - norm)
            if dy == 0 and dx == 0:
                d = jnp.full((H, W), -1.0, dtype=jnp.float32)
            diffs.append(d)
            yy = ys + dy
            xx = xs + dx
            ok = (yy >= 0) & (yy < H) & (xx >= 0) & (xx < W)
            lins.append(jnp.where(ok, yy * W + xx, ZERO_ROW))

    big = jnp.float32(jnp.inf)
    for j in range(NSEL):
        m = functools.reduce(jnp.minimum, diffs)
        found = jnp.zeros((H, W), dtype=jnp.bool_)
        idx_j = jnp.full((H, W), ZERO_ROW, dtype=jnp.int32)
        for i in range(KNN * KNN):
            take = jnp.logical_and(jnp.logical_not(found), diffs[i] == m)
            found = jnp.logical_or(found, take)
            idx_j = jnp.where(take, lins[i], idx_j)
            diffs[i] = jnp.where(take, big, diffs[i])
        idx_ref[j, :, :] = idx_j


def _topk_indices(ch0, mean, std):
    ms = jnp.stack([mean[0], std[0]]).reshape(1, 2)
    return pl.pallas_call(
        _topk_kernel,
        out_shape=jax.ShapeDtypeStruct((NSEL, H, W), jnp.int32),
        in_specs=[
            pl.BlockSpec(memory_space=pltpu.VMEM),
            pl.BlockSpec(memory_space=pltpu.SMEM),
        ],
        out_specs=pl.BlockSpec(memory_space=pltpu.VMEM),
    )(ch0, ms)


TT = 512
TROWS = L + TT
NTT = TROWS // TT


def _transpose_kernel(x_ref, t_ref):
    i = pl.program_id(0)

    @pl.when(i < NTT - 1)
    def _():
        t_ref[...] = x_ref[...].T.astype(jnp.bfloat16)

    @pl.when(i == NTT - 1)
    def _():
        t_ref[...] = jnp.zeros((TT, C), jnp.bfloat16)


def _make_table(x2d):
    return pl.pallas_call(
        _transpose_kernel,
        grid=(NTT,),
        in_specs=[
            pl.BlockSpec((C, TT), lambda i: (0, jnp.minimum(i, NTT - 2))),
        ],
        out_specs=pl.BlockSpec((TT, C), lambda i: (i, 0)),
        out_shape=jax.ShapeDtypeStruct((TROWS, C), jnp.bfloat16),
    )(x2d)


HALF = L // 2
NROWS_H = HALF * NSEL
NCORES = 2
NSUB = 16
NWORKERS = NCORES * NSUB
ROWS_PER_W = NROWS_H // NWORKERS
CHUNK = 1008
NCHUNK = ROWS_PER_W // CHUNK


def _gather_body(table_hbm, idx_hbm, out_hbm, idx_v, rows0, rows1,
                 sem0, sem1):
    wid = lax.axis_index("s") * NCORES + lax.axis_index("c")
    base = wid * ROWS_PER_W
    pltpu.sync_copy(idx_hbm.at[pl.ds(base, ROWS_PER_W)], idx_v)
    bufs = (rows0, rows1)
    sems = (sem0, sem1)

    def start(k, b):
        pltpu.async_copy(
            table_hbm.at[idx_v.at[pl.ds(k * CHUNK, CHUNK)]], bufs[b],
            sems[b])

    def wait(b):
        pltpu.make_async_copy(
            table_hbm.at[idx_v.at[pl.ds(0, CHUNK)]], bufs[b],
            sems[b]).wait()

    def put(k, b):
        pltpu.sync_copy(bufs[b], out_hbm.at[pl.ds(base + k * CHUNK, CHUNK)])

    start(0, 0)

    def pair(p, carry):
        k0 = 2 * p

        start(k0 + 1, 1)
        wait(0)
        put(k0, 0)

        @pl.when(k0 + 2 < NCHUNK)
        def _():
            start(k0 + 2, 0)

        wait(1)
        put(k0 + 1, 1)
        return carry

    lax.fori_loop(0, NCHUNK // 2, pair, 0)
    if NCHUNK % 2 == 1:
        wait(0)
        put(NCHUNK - 1, 0)


@functools.cache
def _make_sc_gather():
    return pl.kernel(
        _gather_body,
        out_type=jax.ShapeDtypeStruct((NROWS_H, C), jnp.bfloat16),
        mesh=plsc.VectorSubcoreMesh(core_axis_name="c", subcore_axis_name="s"),
        scratch_types=[
            pltpu.VMEM((ROWS_PER_W,), jnp.int32),
            pltpu.VMEM((CHUNK, C), jnp.bfloat16),
            pltpu.VMEM((CHUNK, C), jnp.bfloat16),
            pltpu.SemaphoreType.DMA,
            pltpu.SemaphoreType.DMA,
        ],
        compiler_params=pltpu.CompilerParams(use_tc_tiling_on_sc=False),
    )


LT = 512
NT = L // LT


NTH = NT // 2


def _matmul_kernel(g1_ref, g2_ref, w_ref, out_ref):
    i = pl.program_id(0)

    @pl.when(i < NTH)
    def _():
        out_ref[...] = jnp.dot(g1_ref[...], w_ref[...],
                               preferred_element_type=jnp.float32)

    @pl.when(i >= NTH)
    def _():
        out_ref[...] = jnp.dot(g2_ref[...], w_ref[...],
                               preferred_element_type=jnp.float32)


def _matmul(g1_flat, g2_flat, w2t_bf16):
    return pl.pallas_call(
        _matmul_kernel,
        grid=(NT,),
        in_specs=[
            pl.BlockSpec((LT, NSEL * C),
                         lambda i: (jnp.minimum(i, NTH - 1), 0)),
            pl.BlockSpec((LT, NSEL * C),
                         lambda i: (jnp.maximum(i - NTH, 0), 0)),
            pl.BlockSpec((NSEL * C, OC), lambda i: (0, 0)),
        ],
        out_specs=pl.BlockSpec((LT, OC), lambda i: (i, 0)),
        out_shape=jax.ShapeDtypeStruct((L, OC), jnp.float32),
    )(g1_flat, g2_flat, w2t_bf16)


def kernel(input, mean, std, pconv_weight):
    x = input.reshape(C, L)
    table = _make_table(x)

    idx9 = _topk_indices(input[0, 0], mean, std)
    idx9 = idx9.reshape(NSEL, L)
    fi1 = idx9[:, :HALF].T.reshape(NROWS_H)
    fi2 = idx9[:, HALF:].T.reshape(NROWS_H)

    gather = _make_sc_gather()
    g1 = gather(table, fi1).reshape(HALF, NSEL * C)
    g2 = gather(table, fi2).reshape(HALF, NSEL * C)

    w_flat = pconv_weight.reshape(OC, C * NSEL)
    w2t = w_flat.reshape(OC, C, NSEL).transpose(2, 1, 0).reshape(NSEL * C, OC)
    out_rows = _matmul(g1, g2, w2t.astype(jnp.bfloat16))
    return out_rows.T.reshape(1, OC, H, W)
